# baseline (speedup 1.0000x reference)
import functools

import jax
import jax.numpy as jnp
from jax import lax
from jax.experimental import pallas as pl
from jax.experimental.pallas import tpu as pltpu
from jax.experimental.pallas import tpu_sc as plsc

N = 10000
E = 320000
F_IN = 128
H = 16

NC = 2
NS = 16
CB = 128
TCH = E // CB
NB = 8

T_CORE = (TCH // 2, TCH - TCH // 2)
CHMAX = max(-(-T_CORE[0] // NS), -(-T_CORE[1] // NS))

NPAD = 128 * (N // 128 + 1)
RPS = NPAD // NS
P = NPAD // 8
PX = N // 8

_MESH = plsc.VectorSubcoreMesh(core_axis_name="c", subcore_axis_name="s")
_F32 = jnp.float32
_SC_PARAMS = pltpu.CompilerParams(use_tc_tiling_on_sc=False)


def _worker_span(c, s):
    t0, t1 = T_CORE
    tot = jnp.where(c == 0, t0, t1)
    base = jnp.where(c == 0, 0, t0)
    per = tot // NS
    rem = tot - per * NS
    cnt = per + jnp.where(s < rem, 1, 0)
    start = base + s * per + jnp.minimum(s, rem)
    return start, cnt


def _zero_rows(buf, n):
    def body(i, _):
        buf[i] = jnp.zeros((16,), _F32)
        return 0
    lax.fori_loop(0, n, body, 0)


@functools.partial(
    pl.kernel,
    mesh=_MESH,
    compiler_params=_SC_PARAMS,
    out_type=jax.ShapeDtypeStruct((NC, NPAD, H), _F32),
    scratch_types=[
        pltpu.VMEM((CHMAX, CB), jnp.int32),
        pltpu.VMEM((CB, H), _F32),
        pltpu.VMEM((RPS, H), _F32),
        pltpu.VMEM_SHARED((NPAD, H), _F32),
    ],
)
def _deg_kernel(col_hbm, out_hbm, col_v, ones_v, obuf_v, acc_sh):
    c = lax.axis_index("c")
    s = lax.axis_index("s")
    start, cnt = _worker_span(c, s)
    base = jnp.minimum(start, TCH - CHMAX)
    off = start - base
    pltpu.sync_copy(col_hbm.at[pl.ds(base, CHMAX)], col_v)

    def fill(i, _):
        ones_v[i] = jnp.ones((16,), _F32)
        return 0
    lax.fori_loop(0, CB, fill, 0)
    _zero_rows(obuf_v, RPS)
    pltpu.sync_copy(obuf_v, acc_sh.at[pl.ds(s * RPS, RPS)])
    plsc.subcore_barrier()

    def body(j, _):
        pltpu.sync_copy(ones_v, acc_sh.at[col_v.at[off + j]], add=True)
        return 0
    lax.fori_loop(0, cnt, body, 0)
    plsc.subcore_barrier()

    pltpu.sync_copy(acc_sh.at[pl.ds(s * RPS, RPS)], obuf_v)
    pltpu.sync_copy(obuf_v, out_hbm.at[c, pl.ds(s * RPS, RPS)])


@functools.partial(
    pl.kernel,
    mesh=_MESH,
    compiler_params=_SC_PARAMS,
    out_type=jax.ShapeDtypeStruct((NC, NPAD, H), _F32),
    scratch_types=[
        pltpu.VMEM((CHMAX, CB), jnp.int32),
        pltpu.VMEM((CHMAX, CB), jnp.int32),
        [pltpu.VMEM((CB, H), _F32)] * NB,
        pltpu.VMEM((RPS, H), _F32),
        pltpu.VMEM_SHARED((NPAD, H), _F32),
        [pltpu.SemaphoreType.DMA] * NB,
        [pltpu.SemaphoreType.DMA] * NB,
    ],
)
def _agg_kernel(row_hbm, col_hbm, g_hbm, out_hbm,
                row_v, col_v, gbufs, obuf_v, acc_sh, gsems, ssems):
    c = lax.axis_index("c")
    s = lax.axis_index("s")
    start, cnt = _worker_span(c, s)
    base = jnp.minimum(start, TCH - CHMAX)
    off = start - base
    pltpu.sync_copy(row_hbm.at[pl.ds(base, CHMAX)], row_v)
    pltpu.sync_copy(col_hbm.at[pl.ds(base, CHMAX)], col_v)
    _zero_rows(obuf_v, RPS)
    pltpu.sync_copy(obuf_v, acc_sh.at[pl.ds(s * RPS, RPS)])
    plsc.subcore_barrier()

    for b in range(NB):
        pltpu.async_copy(g_hbm.at[row_v.at[off + b]], gbufs[b], gsems[b])

    def body(i, _):
        for b in range(NB):
            j = i * NB + b

            @pl.when(j < cnt)
            def _():
                pltpu.make_async_copy(
                    g_hbm.at[row_v.at[off + j]], gbufs[b], gsems[b]).wait()
                pltpu.async_copy(
                    gbufs[b], acc_sh.at[col_v.at[off + j]], ssems[b], add=True)
        for b in range(NB):
            j = i * NB + b

            @pl.when(j + NB < cnt)
            def _():
                pltpu.make_async_copy(
                    gbufs[b], acc_sh.at[col_v.at[off + j]], ssems[b]).wait()
                pltpu.async_copy(
                    g_hbm.at[row_v.at[off + j + NB]], gbufs[b], gsems[b])
        return 0
    lax.fori_loop(0, (cnt + NB - 1) // NB, body, 0)

    for b in range(NB):
        @pl.when(cnt - NB + b >= 0)
        def _():
            pltpu.make_async_copy(
                gbufs[0], acc_sh.at[col_v.at[off]], ssems[b]).wait()
    plsc.subcore_barrier()

    pltpu.sync_copy(acc_sh.at[pl.ds(s * RPS, RPS)], obuf_v)
    pltpu.sync_copy(obuf_v, out_hbm.at[c, pl.ds(s * RPS, RPS)])


def _mm1_body(x3_ref, w1c_ref, h1_ref):
    acc = jnp.zeros((PX, 128), _F32)
    for a in range(8):
        acc += jnp.dot(x3_ref[:, a, :], w1c_ref[a],
                       preferred_element_type=_F32)
    h1_ref[...] = acc


_mm1 = pl.pallas_call(
    _mm1_body,
    out_shape=jax.ShapeDtypeStruct((PX, 128), _F32),
)


def _dense1_body(deg2_ref, h1_ref, dis_ref, g1_ref):
    deg = deg2_ref[0] + deg2_ref[1] + 1.0
    dis = lax.rsqrt(deg)
    dis_ref[...] = dis
    g1_ref[0:PX, :] = h1_ref[...] * dis[0:PX, :]
    g1_ref[PX:P, :] = jnp.zeros((P - PX, 128), _F32)


_dense1 = pl.pallas_call(
    _dense1_body,
    out_shape=(
        jax.ShapeDtypeStruct((P, 128), _F32),
        jax.ShapeDtypeStruct((P, 128), _F32),
    ),
)


def _dense2_body(agg_ref, g_ref, dis_ref, b_ref, ws_ref, gout_ref):
    s = agg_ref[0] + agg_ref[1] + g_ref[...]
    out = jnp.maximum(dis_ref[...] * s + b_ref[...], 0.0)
    h = jnp.dot(out, ws_ref[...], preferred_element_type=_F32)
    gout_ref[...] = h * dis_ref[...]


_dense2 = pl.pallas_call(
    _dense2_body,
    out_shape=jax.ShapeDtypeStruct((P, 128), _F32),
)


def _dense3_body(agg_ref, g_ref, dis_ref, b_ref, wls_ref, bl_ref, y_ref):
    s = agg_ref[0] + agg_ref[1] + g_ref[...]
    out = jnp.maximum(dis_ref[...] * s + b_ref[...], 0.0)
    y_ref[...] = jnp.dot(out, wls_ref[...], preferred_element_type=_F32) + bl_ref[...]


_dense3 = pl.pallas_call(
    _dense3_body,
    out_shape=jax.ShapeDtypeStruct((P, 8), _F32),
)


@jax.jit
def kernel(x, edge_index, W1, b1, W2, b2, Wl, bl):
    rowc = edge_index[0].astype(jnp.int32).reshape(TCH, CB)
    colc = edge_index[1].astype(jnp.int32).reshape(TCH, CB)
    x3 = x.reshape(PX, 8, F_IN)

    h1 = W1.shape[1]
    eye8 = jnp.eye(8, dtype=_F32)
    W1p = jnp.zeros((F_IN, H), _F32).at[:, :h1].set(W1)
    W1c = jnp.zeros((8, F_IN, 128), _F32)
    for a in range(8):
        W1c = W1c.at[a, :, a * H:a * H + h1].set(W1)
    W2p = jnp.zeros((H, H), _F32).at[:h1, :W2.shape[1]].set(W2)
    W2s = jnp.kron(eye8, W2p)
    Wlp = jnp.zeros((H, 1), _F32).at[:Wl.shape[0]].set(Wl)
    Wls = jnp.kron(eye8, Wlp)
    b1t = jnp.tile(jnp.zeros((1, H), _F32).at[0, :h1].set(b1), (1, 8))
    b2t = jnp.tile(jnp.zeros((1, H), _F32).at[0, :W2.shape[1]].set(b2), (1, 8))
    blt = jnp.tile(bl.reshape(1, 1), (1, 8))

    h1p = _mm1(x3, W1c)
    deg2 = _deg_kernel(colc).reshape(NC, P, 128)
    dis, g1 = _dense1(deg2, h1p)
    agg1 = _agg_kernel(rowc, colc, g1.reshape(NPAD, H)).reshape(NC, P, 128)
    g2 = _dense2(agg1, g1, dis, b1t, W2s)
    agg2 = _agg_kernel(rowc, colc, g2.reshape(NPAD, H)).reshape(NC, P, 128)
    y8 = _dense3(agg2, g2, dis, b2t, Wls, blt)
    return y8.reshape(NPAD)[:N]


# speedup vs baseline: 1.1373x; 1.1373x over previous
import functools

import jax
import jax.numpy as jnp
from jax import lax
from jax.experimental import pallas as pl
from jax.experimental.pallas import tpu as pltpu
from jax.experimental.pallas import tpu_sc as plsc

N = 10000
E = 320000
F_IN = 128
H = 16

NC = 2
NS = 16
CB = 128
TCH = E // CB
NB = 8

T_CORE = (TCH // 2, TCH - TCH // 2)
CHMAX = max(-(-T_CORE[0] // NS), -(-T_CORE[1] // NS))

NPAD = 128 * (N // 128 + 1)
RPS = NPAD // NS
P = NPAD // 8
PX = N // 8

_MESH = plsc.VectorSubcoreMesh(core_axis_name="c", subcore_axis_name="s")
_F32 = jnp.float32
_SC_PARAMS = pltpu.CompilerParams(use_tc_tiling_on_sc=False)


def _worker_span(c, s):
    t0, t1 = T_CORE
    tot = jnp.where(c == 0, t0, t1)
    base = jnp.where(c == 0, 0, t0)
    per = tot // NS
    rem = tot - per * NS
    cnt = per + jnp.where(s < rem, 1, 0)
    start = base + s * per + jnp.minimum(s, rem)
    return start, cnt


def _zero_rows(buf, n):
    def body(i, _):
        buf[i] = jnp.zeros((16,), _F32)
        return 0
    lax.fori_loop(0, n, body, 0)


@functools.partial(
    pl.kernel,
    mesh=_MESH,
    compiler_params=_SC_PARAMS,
    out_type=jax.ShapeDtypeStruct((NC, NPAD, H), _F32),
    scratch_types=[
        pltpu.VMEM((CHMAX, 2, CB), jnp.int32),
        pltpu.VMEM((CB, H), _F32),
        pltpu.VMEM((RPS, H), _F32),
        pltpu.VMEM_SHARED((NPAD, H), _F32),
    ],
)
def _deg_kernel(ei_hbm, out_hbm, ei_v, ones_v, obuf_v, acc_sh):
    c = lax.axis_index("c")
    s = lax.axis_index("s")
    start, cnt = _worker_span(c, s)
    base = jnp.minimum(start, TCH - CHMAX)
    off = start - base
    pltpu.sync_copy(ei_hbm.at[pl.ds(base, CHMAX)], ei_v)

    def fill(i, _):
        ones_v[i] = jnp.ones((16,), _F32)
        return 0
    lax.fori_loop(0, CB, fill, 0)
    _zero_rows(obuf_v, RPS)
    pltpu.sync_copy(obuf_v, acc_sh.at[pl.ds(s * RPS, RPS)])
    plsc.subcore_barrier()

    def body(j, _):
        pltpu.sync_copy(ones_v, acc_sh.at[ei_v.at[off + j, 1]], add=True)
        return 0
    lax.fori_loop(0, cnt, body, 0)
    plsc.subcore_barrier()

    pltpu.sync_copy(acc_sh.at[pl.ds(s * RPS, RPS)], obuf_v)
    pltpu.sync_copy(obuf_v, out_hbm.at[c, pl.ds(s * RPS, RPS)])


@functools.partial(
    pl.kernel,
    mesh=_MESH,
    compiler_params=_SC_PARAMS,
    out_type=jax.ShapeDtypeStruct((NC, NPAD, H), _F32),
    scratch_types=[
        pltpu.VMEM((CHMAX, 2, CB), jnp.int32),
        [pltpu.VMEM((CB, H), _F32)] * NB,
        pltpu.VMEM((RPS, H), _F32),
        pltpu.VMEM_SHARED((NPAD, H), _F32),
        [pltpu.SemaphoreType.DMA] * NB,
        [pltpu.SemaphoreType.DMA] * NB,
    ],
)
def _agg_kernel(ei_hbm, g_hbm, out_hbm,
                ei_v, gbufs, obuf_v, acc_sh, gsems, ssems):
    c = lax.axis_index("c")
    s = lax.axis_index("s")
    start, cnt = _worker_span(c, s)
    base = jnp.minimum(start, TCH - CHMAX)
    off = start - base
    pltpu.sync_copy(ei_hbm.at[pl.ds(base, CHMAX)], ei_v)
    _zero_rows(obuf_v, RPS)
    pltpu.sync_copy(obuf_v, acc_sh.at[pl.ds(s * RPS, RPS)])
    plsc.subcore_barrier()

    for b in range(NB):
        pltpu.async_copy(g_hbm.at[ei_v.at[off + b, 0]], gbufs[b], gsems[b])

    def body(i, _):
        for b in range(NB):
            j = i * NB + b

            @pl.when(j < cnt)
            def _():
                pltpu.make_async_copy(
                    g_hbm.at[ei_v.at[off + j, 0]], gbufs[b], gsems[b]).wait()
                pltpu.async_copy(
                    gbufs[b], acc_sh.at[ei_v.at[off + j, 1]], ssems[b], add=True)
        for b in range(NB):
            j = i * NB + b

            @pl.when(j + NB < cnt)
            def _():
                pltpu.make_async_copy(
                    gbufs[b], acc_sh.at[ei_v.at[off + j, 1]], ssems[b]).wait()
                pltpu.async_copy(
                    g_hbm.at[ei_v.at[off + j + NB, 0]], gbufs[b], gsems[b])
        return 0
    lax.fori_loop(0, (cnt + NB - 1) // NB, body, 0)

    for b in range(NB):
        @pl.when(cnt - NB + b >= 0)
        def _():
            pltpu.make_async_copy(
                gbufs[0], acc_sh.at[ei_v.at[off, 1]], ssems[b]).wait()
    plsc.subcore_barrier()

    pltpu.sync_copy(acc_sh.at[pl.ds(s * RPS, RPS)], obuf_v)
    pltpu.sync_copy(obuf_v, out_hbm.at[c, pl.ds(s * RPS, RPS)])


def _mm1_body(x3_ref, w1c_ref, h1_ref):
    acc = jnp.zeros((PX, 128), _F32)
    for a in range(8):
        acc += jnp.dot(x3_ref[:, a, :], w1c_ref[a],
                       preferred_element_type=_F32)
    h1_ref[...] = acc


_mm1 = pl.pallas_call(
    _mm1_body,
    out_shape=jax.ShapeDtypeStruct((PX, 128), _F32),
)


def _dense1_body(deg2_ref, h1_ref, dis_ref, g1_ref):
    deg = deg2_ref[0] + deg2_ref[1] + 1.0
    dis = lax.rsqrt(deg)
    dis_ref[...] = dis
    g1_ref[0:PX, :] = h1_ref[...] * dis[0:PX, :]
    g1_ref[PX:P, :] = jnp.zeros((P - PX, 128), _F32)


_dense1 = pl.pallas_call(
    _dense1_body,
    out_shape=(
        jax.ShapeDtypeStruct((P, 128), _F32),
        jax.ShapeDtypeStruct((P, 128), _F32),
    ),
)


def _dense2_body(agg_ref, g_ref, dis_ref, b_ref, ws_ref, gout_ref):
    s = agg_ref[0] + agg_ref[1] + g_ref[...]
    out = jnp.maximum(dis_ref[...] * s + b_ref[...], 0.0)
    h = jnp.dot(out, ws_ref[...], preferred_element_type=_F32)
    gout_ref[...] = h * dis_ref[...]


_dense2 = pl.pallas_call(
    _dense2_body,
    out_shape=jax.ShapeDtypeStruct((P, 128), _F32),
)


def _dense3_body(agg_ref, g_ref, dis_ref, b_ref, wls_ref, bl_ref, y_ref):
    s = agg_ref[0] + agg_ref[1] + g_ref[...]
    out = jnp.maximum(dis_ref[...] * s + b_ref[...], 0.0)
    y_ref[...] = jnp.dot(out, wls_ref[...], preferred_element_type=_F32) + bl_ref[...]


_dense3 = pl.pallas_call(
    _dense3_body,
    out_shape=jax.ShapeDtypeStruct((P, 8), _F32),
)


@jax.jit
def kernel(x, edge_index, W1, b1, W2, b2, Wl, bl):
    ei3 = jnp.transpose(
        edge_index.astype(jnp.int32).reshape(2, TCH, CB), (1, 0, 2))
    x3 = x.reshape(PX, 8, F_IN)

    h1 = W1.shape[1]
    eye8 = jnp.eye(8, dtype=_F32)
    W1p = jnp.zeros((F_IN, H), _F32).at[:, :h1].set(W1)
    W1c = jnp.zeros((8, F_IN, 128), _F32)
    for a in range(8):
        W1c = W1c.at[a, :, a * H:a * H + h1].set(W1)
    W2p = jnp.zeros((H, H), _F32).at[:h1, :W2.shape[1]].set(W2)
    W2s = jnp.kron(eye8, W2p)
    Wlp = jnp.zeros((H, 1), _F32).at[:Wl.shape[0]].set(Wl)
    Wls = jnp.kron(eye8, Wlp)
    b1t = jnp.tile(jnp.zeros((1, H), _F32).at[0, :h1].set(b1), (1, 8))
    b2t = jnp.tile(jnp.zeros((1, H), _F32).at[0, :W2.shape[1]].set(b2), (1, 8))
    blt = jnp.tile(bl.reshape(1, 1), (1, 8))

    h1p = _mm1(x3, W1c)
    deg2 = _deg_kernel(ei3).reshape(NC, P, 128)
    dis, g1 = _dense1(deg2, h1p)
    agg1 = _agg_kernel(ei3, g1.reshape(NPAD, H)).reshape(NC, P, 128)
    g2 = _dense2(agg1, g1, dis, b1t, W2s)
    agg2 = _agg_kernel(ei3, g2.reshape(NPAD, H)).reshape(NC, P, 128)
    y8 = _dense3(agg2, g2, dis, b2t, Wls, blt)
    return y8.reshape(NPAD)[:N]
